# Initial kernel scaffold; baseline (speedup 1.0000x reference)
import jax
import jax.numpy as jnp
from jax.experimental import pallas as pl


def kernel(input_ids, table):
    raise NotImplementedError("write your pallas kernel here")


# speedup vs baseline: 2.9663x; 2.9663x over previous
import jax
import jax.numpy as jnp
from jax import lax
from jax.experimental import pallas as pl
from jax.experimental.pallas import tpu as pltpu, tpu_sc as plsc

HIDDEN = 128

_NC = 2
_NS = 16
_NW = _NC * _NS

_B = 4096 * 50
_CHUNK = 128
_ROWS_PER_W = _B // _NW
_CHUNKS_PER_W = _ROWS_PER_W // _CHUNK


def _gather_body(table_hbm, idx_hbm, out_hbm, idx_v, rows_v, sem):
    wid = lax.axis_index("s") * _NC + lax.axis_index("c")
    base = wid * _ROWS_PER_W
    pltpu.sync_copy(idx_hbm.at[pl.ds(base, _ROWS_PER_W)], idx_v)

    def body(j, carry):
        idx_chunk = idx_v.at[pl.ds(j * _CHUNK, _CHUNK)]
        pltpu.async_copy(table_hbm.at[idx_chunk], rows_v, sem).wait()
        pltpu.sync_copy(rows_v, out_hbm.at[pl.ds(base + j * _CHUNK, _CHUNK)])
        return carry

    lax.fori_loop(0, _CHUNKS_PER_W, body, 0)


def kernel(input_ids, table):
    idx_flat = input_ids.reshape(_B)
    mesh = plsc.VectorSubcoreMesh(core_axis_name="c", subcore_axis_name="s")
    out = pl.kernel(
        _gather_body,
        mesh=mesh,
        out_type=jax.ShapeDtypeStruct((_B, HIDDEN), jnp.float32),
        scratch_types=[
            pltpu.VMEM((_ROWS_PER_W,), jnp.int32),
            pltpu.VMEM((_CHUNK, HIDDEN), jnp.float32),
            pltpu.SemaphoreType.DMA,
        ],
    )(table, idx_flat)
    return out.reshape(input_ids.shape + (HIDDEN,))


# speedup vs baseline: 3.3500x; 1.1294x over previous
import jax
import jax.numpy as jnp
from jax import lax
from jax.experimental import pallas as pl
from jax.experimental.pallas import tpu as pltpu, tpu_sc as plsc

HIDDEN = 128

_NC = 2
_NS = 16
_NW = _NC * _NS

_B = 4096 * 50
_CHUNK = 128
_ROWS_PER_W = _B // _NW
_NCHUNK = _ROWS_PER_W // _CHUNK
_NBUF = 5
_LOOK = 2


def _gather_body(table_hbm, idx_hbm, out_hbm, idx_v, rows_v, g_sem, w_sem):
    wid = lax.axis_index("s") * _NC + lax.axis_index("c")
    base = wid * _ROWS_PER_W
    pltpu.sync_copy(idx_hbm.at[pl.ds(base, _ROWS_PER_W)], idx_v)

    def start_gather(c, b):
        idx_chunk = idx_v.at[pl.ds(c * _CHUNK, _CHUNK)]
        pltpu.async_copy(table_hbm.at[idx_chunk], rows_v.at[b], g_sem.at[b])

    def wait_gather(b):
        pltpu.make_async_copy(
            table_hbm.at[pl.ds(0, _CHUNK)], rows_v.at[b], g_sem.at[b]
        ).wait()

    def start_write(c, b):
        pltpu.async_copy(
            rows_v.at[b], out_hbm.at[pl.ds(base + c * _CHUNK, _CHUNK)], w_sem.at[b]
        )

    def wait_write(b):
        pltpu.make_async_copy(
            rows_v.at[b], out_hbm.at[pl.ds(base, _CHUNK)], w_sem.at[b]
        ).wait()

    def visit(c, has_prev_write, do_look):
        if has_prev_write:
            wait_write((c + _LOOK) % _NBUF)
        if do_look:
            start_gather(c + _LOOK, (c + _LOOK) % _NBUF)
        wait_gather(c % _NBUF)
        start_write(c, c % _NBUF)

    for c in range(_LOOK):
        start_gather(c, c % _NBUF)

    for c in range(_NBUF):
        visit(c, has_prev_write=(c >= _NBUF - _LOOK), do_look=True)

    def body(g, carry):
        for bi in range(_NBUF):
            c = g * _NBUF + bi
            visit(c, has_prev_write=True, do_look=True)
        return carry

    lax.fori_loop(1, _NCHUNK // _NBUF - 1, body, 0)

    for bi in range(_NBUF):
        c = (_NCHUNK // _NBUF - 1) * _NBUF + bi
        visit(c, has_prev_write=True, do_look=(c + _LOOK < _NCHUNK))

    for c in range(_NCHUNK - (_NBUF - _LOOK), _NCHUNK):
        wait_write(c % _NBUF)


def kernel(input_ids, table):
    idx_flat = input_ids.reshape(_B)
    mesh = plsc.VectorSubcoreMesh(core_axis_name="c", subcore_axis_name="s")
    out = pl.kernel(
        _gather_body,
        mesh=mesh,
        out_type=jax.ShapeDtypeStruct((_B, HIDDEN), jnp.float32),
        scratch_types=[
            pltpu.VMEM((_ROWS_PER_W,), jnp.int32),
            pltpu.VMEM((_NBUF, _CHUNK, HIDDEN), jnp.float32),
            pltpu.SemaphoreType.DMA((_NBUF,)),
            pltpu.SemaphoreType.DMA((_NBUF,)),
        ],
    )(table, idx_flat)
    return out.reshape(input_ids.shape + (HIDDEN,))


# speedup vs baseline: 5.9366x; 1.7721x over previous
import jax
import jax.numpy as jnp
from jax import lax
from jax.experimental import pallas as pl
from jax.experimental.pallas import tpu as pltpu, tpu_sc as plsc

HIDDEN = 128

_NC = 2
_NS = 16
_NW = _NC * _NS

_SEQS = 4096
_SEQLEN = 50
_SEQ_PER_W = _SEQS // _NW
_NBUF = 8
_LOOK = 4


def _gather_body(table_hbm, idx_hbm, out_hbm, idx_v, rows_v, g_sem, w_sem):
    wid = lax.axis_index("s") * _NC + lax.axis_index("c")
    base = wid * _SEQ_PER_W
    pltpu.sync_copy(idx_hbm.at[pl.ds(base, _SEQ_PER_W)], idx_v)

    def start_gather(c, b):
        pltpu.async_copy(table_hbm.at[idx_v.at[c]], rows_v.at[b], g_sem.at[b])

    def wait_gather(b):
        pltpu.make_async_copy(
            table_hbm.at[idx_v.at[0]], rows_v.at[b], g_sem.at[b]
        ).wait()

    def start_write(c, b):
        pltpu.async_copy(rows_v.at[b], out_hbm.at[base + c], w_sem.at[b])

    def wait_write(b):
        pltpu.make_async_copy(rows_v.at[b], out_hbm.at[base], w_sem.at[b]).wait()

    def visit(c, has_prev_write, do_look):
        if has_prev_write:
            wait_write((c + _LOOK) % _NBUF)
        if do_look:
            start_gather(c + _LOOK, (c + _LOOK) % _NBUF)
        wait_gather(c % _NBUF)
        start_write(c, c % _NBUF)

    for c in range(_LOOK):
        start_gather(c, c % _NBUF)

    for c in range(_NBUF):
        visit(c, has_prev_write=(c >= _NBUF - _LOOK), do_look=True)

    def body(g, carry):
        for bi in range(_NBUF):
            c = g * _NBUF + bi
            visit(c, has_prev_write=True, do_look=True)
        return carry

    lax.fori_loop(1, _SEQ_PER_W // _NBUF - 1, body, 0)

    for bi in range(_NBUF):
        c = (_SEQ_PER_W // _NBUF - 1) * _NBUF + bi
        visit(c, has_prev_write=True, do_look=(c + _LOOK < _SEQ_PER_W))

    for c in range(_SEQ_PER_W - (_NBUF - _LOOK), _SEQ_PER_W):
        wait_write(c % _NBUF)


def kernel(input_ids, table):
    mesh = plsc.VectorSubcoreMesh(core_axis_name="c", subcore_axis_name="s")
    out = pl.kernel(
        _gather_body,
        mesh=mesh,
        out_type=jax.ShapeDtypeStruct((_SEQS, _SEQLEN, HIDDEN), jnp.float32),
        scratch_types=[
            pltpu.VMEM((_SEQ_PER_W, _SEQLEN), jnp.int32),
            pltpu.VMEM((_NBUF, _SEQLEN, HIDDEN), jnp.float32),
            pltpu.SemaphoreType.DMA((_NBUF,)),
            pltpu.SemaphoreType.DMA((_NBUF,)),
        ],
    )(table, input_ids)
    return out


# speedup vs baseline: 10.7686x; 1.8139x over previous
import jax
import jax.numpy as jnp
from jax import lax
from jax.experimental import pallas as pl
from jax.experimental.pallas import tpu as pltpu, tpu_sc as plsc

HIDDEN = 128

_NC = 2
_NS = 16
_NW = _NC * _NS

_SEQS = 4096
_SEQLEN = 50
_SEQ_PER_W = _SEQS // _NW
_NBUF = 5
_LOOK = 2


def _gather_body(table_hbm, idx_hbm, out_hbm, idx_v, rows_v, g_sem, w_sem):
    wid = lax.axis_index("s") * _NC + lax.axis_index("c")
    col = wid * _SEQ_PER_W
    pltpu.sync_copy(idx_hbm.at[:, pl.ds(col, _SEQ_PER_W)], idx_v)

    def start_gather(t, b):
        pltpu.async_copy(table_hbm.at[idx_v.at[t]], rows_v.at[b], g_sem.at[b])

    def wait_gather(b):
        pltpu.make_async_copy(
            table_hbm.at[idx_v.at[0]], rows_v.at[b], g_sem.at[b]
        ).wait()

    def start_write(t, b):
        pltpu.async_copy(
            rows_v.at[b], out_hbm.at[t, pl.ds(col, _SEQ_PER_W)], w_sem.at[b]
        )

    def wait_write(b):
        pltpu.make_async_copy(
            rows_v.at[b], out_hbm.at[0, pl.ds(col, _SEQ_PER_W)], w_sem.at[b]
        ).wait()

    def visit(t, has_prev_write, do_look):
        if has_prev_write:
            wait_write((t + _LOOK) % _NBUF)
        if do_look:
            start_gather(t + _LOOK, (t + _LOOK) % _NBUF)
        wait_gather(t % _NBUF)
        start_write(t, t % _NBUF)

    for t in range(_LOOK):
        start_gather(t, t % _NBUF)

    for t in range(_NBUF):
        visit(t, has_prev_write=(t >= _NBUF - _LOOK), do_look=True)

    def body(g, carry):
        for bi in range(_NBUF):
            t = g * _NBUF + bi
            visit(t, has_prev_write=True, do_look=True)
        return carry

    lax.fori_loop(1, _SEQLEN // _NBUF - 1, body, 0)

    for bi in range(_NBUF):
        t = (_SEQLEN // _NBUF - 1) * _NBUF + bi
        visit(t, has_prev_write=True, do_look=(t + _LOOK < _SEQLEN))

    for t in range(_SEQLEN - (_NBUF - _LOOK), _SEQLEN):
        wait_write(t % _NBUF)


def kernel(input_ids, table):
    ids_t = input_ids.T
    mesh = plsc.VectorSubcoreMesh(core_axis_name="c", subcore_axis_name="s")
    out_t = pl.kernel(
        _gather_body,
        mesh=mesh,
        out_type=jax.ShapeDtypeStruct((_SEQLEN, _SEQS, HIDDEN), jnp.float32),
        scratch_types=[
            pltpu.VMEM((_SEQLEN, _SEQ_PER_W), jnp.int32),
            pltpu.VMEM((_NBUF, _SEQ_PER_W, HIDDEN), jnp.float32),
            pltpu.SemaphoreType.DMA((_NBUF,)),
            pltpu.SemaphoreType.DMA((_NBUF,)),
        ],
    )(table, ids_t)
    return jnp.transpose(out_t, (1, 0, 2))
